# Initial kernel scaffold; baseline (speedup 1.0000x reference)
import jax
import jax.numpy as jnp
from jax.experimental import pallas as pl


def kernel(x, edge_index, W1, b1, Wg, bg, gamma, beta):
    raise NotImplementedError("write your pallas kernel here")


# speedup vs baseline: 26.1648x; 26.1648x over previous
import functools

import jax
import jax.numpy as jnp
from jax import lax
from jax.experimental import pallas as pl
from jax.experimental.pallas import tpu as pltpu
from jax.experimental.pallas import tpu_sc as plsc

NC = 2
NS = 16
NW = NC * NS
CH = 80
RB = 1000


def _mm_body(x_ref, w1_ref, b1_ref, wg_ref, h_ref, xw_ref):
    h = jnp.dot(x_ref[...], w1_ref[...], preferred_element_type=jnp.float32)
    h = jnp.maximum(h + b1_ref[...], 0.0)
    h_ref[...] = h
    xw_ref[...] = jnp.dot(h, wg_ref[...], preferred_element_type=jnp.float32)


def _matmuls(x, W1, b1, Wg):
    n, d_in = x.shape
    d = W1.shape[1]
    return pl.pallas_call(
        _mm_body,
        grid=(n // RB,),
        in_specs=[
            pl.BlockSpec((RB, d_in), lambda i: (i, 0)),
            pl.BlockSpec((d_in, d), lambda i: (0, 0)),
            pl.BlockSpec((1, d), lambda i: (0, 0)),
            pl.BlockSpec((d, d), lambda i: (0, 0)),
        ],
        out_specs=[
            pl.BlockSpec((RB, d), lambda i: (i, 0)),
            pl.BlockSpec((RB, d), lambda i: (i, 0)),
        ],
        out_shape=[
            jax.ShapeDtypeStruct((n, d), jnp.float32),
            jax.ShapeDtypeStruct((n, d), jnp.float32),
        ],
    )(x, W1, b1.reshape(1, d), Wg)


def _make_deg_kernel(n, e):
    nch = e // (NW * CH)
    nzc = n // CH
    kmax = pl.cdiv(nzc, NS)
    mesh = plsc.VectorSubcoreMesh(core_axis_name="c", subcore_axis_name="s")

    @functools.partial(
        pl.kernel,
        out_type=jax.ShapeDtypeStruct((NC * n,), jnp.float32),
        mesh=mesh,
        scratch_types=[
            pltpu.VMEM((nch, CH), jnp.int32),
            pltpu.VMEM((CH,), jnp.float32),
            pltpu.VMEM((CH,), jnp.float32),
            pltpu.VMEM_SHARED((n,), jnp.float32),
        ],
    )
    def deg_kernel(dst_hbm, out_hbm, idx_v, ones_v, zero_v, deg_sp):
        c = lax.axis_index("c")
        s = lax.axis_index("s")
        wid = s * NC + c
        pltpu.sync_copy(dst_hbm.at[wid], idx_v)
        for k in range(CH // 16):
            ones_v[pl.ds(k * 16, 16)] = jnp.full((16,), 1.0, jnp.float32)
            zero_v[pl.ds(k * 16, 16)] = jnp.zeros((16,), jnp.float32)
        for k in range(kmax):
            cid = s + NS * k

            @pl.when(cid < nzc)
            def _():
                pltpu.sync_copy(zero_v, deg_sp.at[pl.ds(cid * CH, CH)])

        plsc.subcore_barrier()

        def body(j, carry):
            pltpu.sync_copy(ones_v, deg_sp.at[idx_v.at[j]], add=True)
            return carry

        lax.fori_loop(0, nch, body, 0)
        plsc.subcore_barrier()
        for k in range(kmax):
            cid = s + NS * k

            @pl.when(cid < nzc)
            def _():
                pltpu.sync_copy(deg_sp.at[pl.ds(cid * CH, CH)], zero_v)
                pltpu.sync_copy(zero_v,
                                out_hbm.at[pl.ds(c * n + cid * CH, CH)])

    return deg_kernel


def _prep_body(degt_ref, xw_ref, xs_ref, dinv_ref):
    deg = jnp.sum(degt_ref[...], axis=1, keepdims=True) + 1.0
    dinv = lax.rsqrt(deg)
    xs_ref[...] = xw_ref[...] * dinv
    dinv_ref[...] = dinv


def _prep(degt, xw):
    n, d = xw.shape
    return pl.pallas_call(
        _prep_body,
        grid=(n // RB,),
        in_specs=[
            pl.BlockSpec((RB, NC), lambda i: (i, 0)),
            pl.BlockSpec((RB, d), lambda i: (i, 0)),
        ],
        out_specs=[
            pl.BlockSpec((RB, d), lambda i: (i, 0)),
            pl.BlockSpec((RB, 1), lambda i: (i, 0)),
        ],
        out_shape=[
            jax.ShapeDtypeStruct((n, d), jnp.float32),
            jax.ShapeDtypeStruct((n, 1), jnp.float32),
        ],
    )(degt, xw)


SPLIT = 5040
GPAD = 64


def _make_mp_kernel(n, d, e):
    nch = e // (NW * CH)
    half = nch // 2
    acc_rows = SPLIT + CH
    nzc = acc_rows // CH
    kmax = pl.cdiv(nzc, NS)
    c0chunks = SPLIT // CH
    mesh = plsc.VectorSubcoreMesh(core_axis_name="c", subcore_axis_name="s")

    @functools.partial(
        pl.kernel,
        out_type=jax.ShapeDtypeStruct((n, d), jnp.float32),
        mesh=mesh,
        scratch_types=[
            pltpu.VMEM((nch, CH), jnp.int32),
            pltpu.VMEM((nch, CH), jnp.int32),
            pltpu.VMEM((CH, d), jnp.float32),
            pltpu.VMEM((CH, d), jnp.float32),
            pltpu.VMEM_SHARED((acc_rows, d), jnp.float32),
            pltpu.SemaphoreType.DMA,
            pltpu.SemaphoreType.DMA,
        ],
    )
    def mp_kernel(xs_hbm, src_hbm, dst_hbm, out_hbm,
                  srcv, dstv, rows0, rows1, acc_sp, sem0, sem1):
        c = lax.axis_index("c")
        s = lax.axis_index("s")

        lo = c * SPLIT
        hi = SPLIT + c * (n - SPLIT)

        def zb(r, carry):
            for k2 in range(d // 16):
                rows0[r, pl.ds(k2 * 16, 16)] = jnp.zeros((16,), jnp.float32)
            return carry

        lax.fori_loop(0, CH, zb, 0)
        for k in range(kmax):
            cid = s + NS * k

            @pl.when(cid < nzc)
            def _():
                pltpu.sync_copy(rows0, acc_sp.at[pl.ds(cid * CH, CH)])

        plsc.subcore_barrier()

        for t in range(NC):
            wid = s * NC + t
            pltpu.sync_copy(src_hbm.at[wid], srcv)
            pltpu.sync_copy(dst_hbm.at[wid], dstv)

            def tf(j, carry):
                for k2 in range(CH // 16):
                    v = dstv[j, pl.ds(k2 * 16, 16)]
                    m = (v >= lo) & (v < hi)
                    g = SPLIT + (v & (GPAD - 1))
                    dstv[j, pl.ds(k2 * 16, 16)] = jnp.where(m, v - lo, g)
                return carry

            lax.fori_loop(0, nch, tf, 0)

            pltpu.async_copy(xs_hbm.at[srcv.at[0]], rows0, sem0)

            def body(i, carry):
                j = 1 + 2 * i
                pltpu.async_copy(xs_hbm.at[srcv.at[j]], rows1, sem1)
                pltpu.make_async_copy(xs_hbm.at[srcv.at[j - 1]], rows0,
                                      sem0).wait()
                pltpu.sync_copy(rows0, acc_sp.at[dstv.at[j - 1]], add=True)
                pltpu.async_copy(xs_hbm.at[srcv.at[j + 1]], rows0, sem0)
                pltpu.make_async_copy(xs_hbm.at[srcv.at[j]], rows1,
                                      sem1).wait()
                pltpu.sync_copy(rows1, acc_sp.at[dstv.at[j]], add=True)
                return carry

            lax.fori_loop(0, half, body, 0)
            pltpu.make_async_copy(xs_hbm.at[srcv.at[nch - 1]], rows0,
                                  sem0).wait()
            pltpu.sync_copy(rows0, acc_sp.at[dstv.at[nch - 1]], add=True)

        plsc.subcore_barrier()
        for k in range(kmax):
            cid = s + NS * k

            @pl.when(cid < c0chunks - c)
            def _():
                pltpu.sync_copy(acc_sp.at[pl.ds(cid * CH, CH)], rows0)
                pltpu.sync_copy(rows0,
                                out_hbm.at[pl.ds(lo + cid * CH, CH)])

    return mp_kernel


def _final_body(h_ref, acc_ref, xs_ref, dinv_ref, bg_ref, g_ref, b_ref,
                out_ref):
    h = h_ref[...]
    t = acc_ref[...] + xs_ref[...]
    h2 = jnp.maximum(t * dinv_ref[...] + bg_ref[...], 0.0)
    cat = jnp.concatenate([h, h2], axis=1)
    mu = jnp.mean(cat, axis=1, keepdims=True)
    xc = cat - mu
    var = jnp.mean(xc * xc, axis=1, keepdims=True)
    out_ref[...] = xc * lax.rsqrt(var + 1e-5) * g_ref[...] + b_ref[...]


def _final(h, acc, xs, dinv, bg, gamma, beta):
    n, d = h.shape
    return pl.pallas_call(
        _final_body,
        grid=(n // RB,),
        in_specs=[
            pl.BlockSpec((RB, d), lambda i: (i, 0)),
            pl.BlockSpec((RB, d), lambda i: (i, 0)),
            pl.BlockSpec((RB, d), lambda i: (i, 0)),
            pl.BlockSpec((RB, 1), lambda i: (i, 0)),
            pl.BlockSpec((1, d), lambda i: (0, 0)),
            pl.BlockSpec((1, 2 * d), lambda i: (0, 0)),
            pl.BlockSpec((1, 2 * d), lambda i: (0, 0)),
        ],
        out_specs=pl.BlockSpec((RB, 2 * d), lambda i: (i, 0)),
        out_shape=jax.ShapeDtypeStruct((n, 2 * d), jnp.float32),
    )(h, acc, xs, dinv, bg.reshape(1, d), gamma.reshape(1, 2 * d),
      beta.reshape(1, 2 * d))


def kernel(x, edge_index, W1, b1, Wg, bg, gamma, beta):
    n, _ = x.shape
    d = W1.shape[1]
    e = edge_index.shape[1]
    nch = e // (NW * CH)
    src3 = edge_index[0].reshape(NW, nch, CH)
    dst3 = edge_index[1].reshape(NW, nch, CH)

    h, xw = _matmuls(x, W1, b1, Wg)
    degs = _make_deg_kernel(n, e)(dst3).reshape(NC, n)
    xs, dinv = _prep(degs.T, xw)
    acc = _make_mp_kernel(n, d, e)(xs, src3, dst3)
    return _final(h, acc, xs, dinv, bg, gamma, beta)


# speedup vs baseline: 31.4900x; 1.2035x over previous
import functools

import jax
import jax.numpy as jnp
from jax import lax
from jax.experimental import pallas as pl
from jax.experimental.pallas import tpu as pltpu
from jax.experimental.pallas import tpu_sc as plsc

NC = 2
NS = 16
NW = NC * NS
CH = 80
RB = 1000


def _mm_body(x_ref, w1_ref, b1_ref, wg_ref, h_ref, xw_ref):
    h = jnp.dot(x_ref[...], w1_ref[...], preferred_element_type=jnp.float32)
    h = jnp.maximum(h + b1_ref[...], 0.0)
    h_ref[...] = h
    xw_ref[...] = jnp.dot(h, wg_ref[...], preferred_element_type=jnp.float32)


def _matmuls(x, W1, b1, Wg):
    n, d_in = x.shape
    d = W1.shape[1]
    return pl.pallas_call(
        _mm_body,
        grid=(n // RB,),
        in_specs=[
            pl.BlockSpec((RB, d_in), lambda i: (i, 0)),
            pl.BlockSpec((d_in, d), lambda i: (0, 0)),
            pl.BlockSpec((1, d), lambda i: (0, 0)),
            pl.BlockSpec((d, d), lambda i: (0, 0)),
        ],
        out_specs=[
            pl.BlockSpec((RB, d), lambda i: (i, 0)),
            pl.BlockSpec((RB, d), lambda i: (i, 0)),
        ],
        out_shape=[
            jax.ShapeDtypeStruct((n, d), jnp.float32),
            jax.ShapeDtypeStruct((n, d), jnp.float32),
        ],
    )(x, W1, b1.reshape(1, d), Wg)


def _make_deg_kernel(n, e):
    nch = e // (NW * CH)
    nzc = n // CH
    kmax = pl.cdiv(nzc, NS)
    mesh = plsc.VectorSubcoreMesh(core_axis_name="c", subcore_axis_name="s")

    @functools.partial(
        pl.kernel,
        out_type=jax.ShapeDtypeStruct((NC * n,), jnp.float32),
        mesh=mesh,
        scratch_types=[
            pltpu.VMEM((nch, CH), jnp.int32),
            pltpu.VMEM((CH,), jnp.float32),
            pltpu.VMEM((CH,), jnp.float32),
            pltpu.VMEM_SHARED((n,), jnp.float32),
        ],
    )
    def deg_kernel(dst_hbm, out_hbm, idx_v, ones_v, zero_v, deg_sp):
        c = lax.axis_index("c")
        s = lax.axis_index("s")
        wid = s * NC + c
        pltpu.sync_copy(dst_hbm.at[wid], idx_v)
        for k in range(CH // 16):
            ones_v[pl.ds(k * 16, 16)] = jnp.full((16,), 1.0, jnp.float32)
            zero_v[pl.ds(k * 16, 16)] = jnp.zeros((16,), jnp.float32)
        for k in range(kmax):
            cid = s + NS * k

            @pl.when(cid < nzc)
            def _():
                pltpu.sync_copy(zero_v, deg_sp.at[pl.ds(cid * CH, CH)])

        plsc.subcore_barrier()

        def body(j, carry):
            pltpu.sync_copy(ones_v, deg_sp.at[idx_v.at[j]], add=True)
            return carry

        lax.fori_loop(0, nch, body, 0)
        plsc.subcore_barrier()
        for k in range(kmax):
            cid = s + NS * k

            @pl.when(cid < nzc)
            def _():
                pltpu.sync_copy(deg_sp.at[pl.ds(cid * CH, CH)], zero_v)
                pltpu.sync_copy(zero_v,
                                out_hbm.at[pl.ds(c * n + cid * CH, CH)])

    return deg_kernel


def _prep_body(degt_ref, xw_ref, xs_ref, dinv_ref):
    deg = jnp.sum(degt_ref[...], axis=1, keepdims=True) + 1.0
    dinv = lax.rsqrt(deg)
    xs_ref[...] = xw_ref[...] * dinv
    dinv_ref[...] = dinv


def _prep(degt, xw):
    n, d = xw.shape
    return pl.pallas_call(
        _prep_body,
        grid=(n // RB,),
        in_specs=[
            pl.BlockSpec((RB, NC), lambda i: (i, 0)),
            pl.BlockSpec((RB, d), lambda i: (i, 0)),
        ],
        out_specs=[
            pl.BlockSpec((RB, d), lambda i: (i, 0)),
            pl.BlockSpec((RB, 1), lambda i: (i, 0)),
        ],
        out_shape=[
            jax.ShapeDtypeStruct((n, d), jnp.float32),
            jax.ShapeDtypeStruct((n, 1), jnp.float32),
        ],
    )(degt, xw)


SPLIT = 5040
GPAD = 64


NBUF = 5


def _make_mp_kernel(n, d, e):
    nch = e // (NW * CH)
    assert nch % NBUF == 0
    acc_rows = SPLIT + CH
    nzc = acc_rows // CH
    kmax = pl.cdiv(nzc, NS)
    c0chunks = SPLIT // CH
    mesh = plsc.VectorSubcoreMesh(core_axis_name="c", subcore_axis_name="s")

    @functools.partial(
        pl.kernel,
        out_type=jax.ShapeDtypeStruct((n, d), jnp.float32),
        mesh=mesh,
        scratch_types=[
            pltpu.VMEM((nch, CH), jnp.int32),
            pltpu.VMEM((nch, CH), jnp.int32),
            [pltpu.VMEM((CH, d), jnp.float32) for _ in range(NBUF)],
            [pltpu.SemaphoreType.DMA for _ in range(NBUF)],
            pltpu.VMEM_SHARED((acc_rows, d), jnp.float32),
        ],
    )
    def mp_kernel(xs_hbm, src_hbm, dst_hbm, out_hbm,
                  srcv, dstv, rows, sems, acc_sp):
        c = lax.axis_index("c")
        s = lax.axis_index("s")
        rows0 = rows[0]

        lo = c * SPLIT
        hi = SPLIT + c * (n - SPLIT)

        def zb(r, carry):
            for k2 in range(d // 16):
                rows0[r, pl.ds(k2 * 16, 16)] = jnp.zeros((16,), jnp.float32)
            return carry

        lax.fori_loop(0, CH, zb, 0)
        for k in range(kmax):
            cid = s + NS * k

            @pl.when(cid < nzc)
            def _():
                pltpu.sync_copy(rows0, acc_sp.at[pl.ds(cid * CH, CH)])

        for t in range(NC):
            wid = s * NC + t
            pltpu.sync_copy(src_hbm.at[wid], srcv)
            pltpu.sync_copy(dst_hbm.at[wid], dstv)

            def tf(j, carry):
                for k2 in range(CH // 16):
                    v = dstv[j, pl.ds(k2 * 16, 16)]
                    m = (v >= lo) & (v < hi)
                    g = SPLIT + (v & (GPAD - 1))
                    dstv[j, pl.ds(k2 * 16, 16)] = jnp.where(m, v - lo, g)
                return carry

            lax.fori_loop(0, nch, tf, 0)

            for k in range(NBUF - 1):
                pltpu.async_copy(xs_hbm.at[srcv.at[k]], rows[k], sems[k])

            def body(i, carry):
                for k in range(NBUF):
                    j = NBUF * i + k
                    pltpu.make_async_copy(xs_hbm.at[srcv.at[j]], rows[k],
                                          sems[k]).wait()
                    pltpu.sync_copy(rows[k], acc_sp.at[dstv.at[j]],
                                    add=True)
                    nxt = j + NBUF - 1
                    kn = (k + NBUF - 1) % NBUF

                    @pl.when(nxt < nch)
                    def _():
                        pltpu.async_copy(xs_hbm.at[srcv.at[nxt]], rows[kn],
                                         sems[kn])
                return carry

            lax.fori_loop(0, nch // NBUF, body, 0)

        plsc.subcore_barrier()
        for k in range(kmax):
            cid = s + NS * k

            @pl.when(cid < c0chunks - c)
            def _():
                pltpu.sync_copy(acc_sp.at[pl.ds(cid * CH, CH)], rows0)
                pltpu.sync_copy(rows0,
                                out_hbm.at[pl.ds(lo + cid * CH, CH)])

    return mp_kernel


def _final_body(h_ref, acc_ref, xs_ref, dinv_ref, bg_ref, g_ref, b_ref,
                out_ref):
    h = h_ref[...]
    t = acc_ref[...] + xs_ref[...]
    h2 = jnp.maximum(t * dinv_ref[...] + bg_ref[...], 0.0)
    cat = jnp.concatenate([h, h2], axis=1)
    mu = jnp.mean(cat, axis=1, keepdims=True)
    xc = cat - mu
    var = jnp.mean(xc * xc, axis=1, keepdims=True)
    out_ref[...] = xc * lax.rsqrt(var + 1e-5) * g_ref[...] + b_ref[...]


def _final(h, acc, xs, dinv, bg, gamma, beta):
    n, d = h.shape
    return pl.pallas_call(
        _final_body,
        grid=(n // RB,),
        in_specs=[
            pl.BlockSpec((RB, d), lambda i: (i, 0)),
            pl.BlockSpec((RB, d), lambda i: (i, 0)),
            pl.BlockSpec((RB, d), lambda i: (i, 0)),
            pl.BlockSpec((RB, 1), lambda i: (i, 0)),
            pl.BlockSpec((1, d), lambda i: (0, 0)),
            pl.BlockSpec((1, 2 * d), lambda i: (0, 0)),
            pl.BlockSpec((1, 2 * d), lambda i: (0, 0)),
        ],
        out_specs=pl.BlockSpec((RB, 2 * d), lambda i: (i, 0)),
        out_shape=jax.ShapeDtypeStruct((n, 2 * d), jnp.float32),
    )(h, acc, xs, dinv, bg.reshape(1, d), gamma.reshape(1, 2 * d),
      beta.reshape(1, 2 * d))


def kernel(x, edge_index, W1, b1, Wg, bg, gamma, beta):
    n, _ = x.shape
    d = W1.shape[1]
    e = edge_index.shape[1]
    nch = e // (NW * CH)
    src3 = edge_index[0].reshape(NW, nch, CH)
    dst3 = edge_index[1].reshape(NW, nch, CH)

    h, xw = _matmuls(x, W1, b1, Wg)
    degs = _make_deg_kernel(n, e)(dst3).reshape(NC, n)
    xs, dinv = _prep(degs.T, xw)
    acc = _make_mp_kernel(n, d, e)(xs, src3, dst3)
    return _final(h, acc, xs, dinv, bg, gamma, beta)


# speedup vs baseline: 31.6856x; 1.0062x over previous
import functools

import jax
import jax.numpy as jnp
from jax import lax
from jax.experimental import pallas as pl
from jax.experimental.pallas import tpu as pltpu
from jax.experimental.pallas import tpu_sc as plsc

NC = 2
NS = 16
NW = NC * NS
CH = 80
RB = 1000


def _mm_body(x_ref, w1_ref, b1_ref, wg_ref, h_ref, xw_ref):
    h = jnp.dot(x_ref[...], w1_ref[...], preferred_element_type=jnp.float32)
    h = jnp.maximum(h + b1_ref[...], 0.0)
    h_ref[...] = h
    xw_ref[...] = jnp.dot(h, wg_ref[...], preferred_element_type=jnp.float32)


def _matmuls(x, W1, b1, Wg):
    n, d_in = x.shape
    d = W1.shape[1]
    return pl.pallas_call(
        _mm_body,
        grid=(n // RB,),
        in_specs=[
            pl.BlockSpec((RB, d_in), lambda i: (i, 0)),
            pl.BlockSpec((d_in, d), lambda i: (0, 0)),
            pl.BlockSpec((1, d), lambda i: (0, 0)),
            pl.BlockSpec((d, d), lambda i: (0, 0)),
        ],
        out_specs=[
            pl.BlockSpec((RB, d), lambda i: (i, 0)),
            pl.BlockSpec((RB, d), lambda i: (i, 0)),
        ],
        out_shape=[
            jax.ShapeDtypeStruct((n, d), jnp.float32),
            jax.ShapeDtypeStruct((n, d), jnp.float32),
        ],
    )(x, W1, b1.reshape(1, d), Wg)


def _make_deg_kernel(n, e):
    nch = e // (NW * CH)
    nzc = n // CH
    kmax = pl.cdiv(nzc, NS)
    mesh = plsc.VectorSubcoreMesh(core_axis_name="c", subcore_axis_name="s")

    @functools.partial(
        pl.kernel,
        out_type=jax.ShapeDtypeStruct((NC * n,), jnp.float32),
        mesh=mesh,
        scratch_types=[
            pltpu.VMEM((nch, CH), jnp.int32),
            pltpu.VMEM((CH,), jnp.float32),
            pltpu.VMEM((CH,), jnp.float32),
            pltpu.VMEM_SHARED((n,), jnp.float32),
            pltpu.SemaphoreType.DMA,
        ],
    )
    def deg_kernel(dst_hbm, out_hbm, idx_v, ones_v, zero_v, deg_sp, sem):
        c = lax.axis_index("c")
        s = lax.axis_index("s")
        wid = s * NC + c
        pltpu.sync_copy(dst_hbm.at[wid], idx_v)
        for k in range(CH // 16):
            ones_v[pl.ds(k * 16, 16)] = jnp.full((16,), 1.0, jnp.float32)
            zero_v[pl.ds(k * 16, 16)] = jnp.zeros((16,), jnp.float32)
        for k in range(kmax):
            cid = s + NS * k

            @pl.when(cid < nzc)
            def _():
                pltpu.sync_copy(zero_v, deg_sp.at[pl.ds(cid * CH, CH)])

        plsc.subcore_barrier()

        def body(j, carry):
            pltpu.async_copy(ones_v, deg_sp.at[idx_v.at[j]], sem, add=True)
            return carry

        lax.fori_loop(0, nch, body, 0)

        def drain(j, carry):
            pltpu.make_async_copy(ones_v, deg_sp.at[idx_v.at[j]],
                                  sem).wait()
            return carry

        lax.fori_loop(0, nch, drain, 0)
        plsc.subcore_barrier()
        for k in range(kmax):
            cid = s + NS * k

            @pl.when(cid < nzc)
            def _():
                pltpu.sync_copy(deg_sp.at[pl.ds(cid * CH, CH)], zero_v)
                pltpu.sync_copy(zero_v,
                                out_hbm.at[pl.ds(c * n + cid * CH, CH)])

    return deg_kernel


def _prep_body(degt_ref, xw_ref, xs_ref, dinv_ref):
    deg = jnp.sum(degt_ref[...], axis=1, keepdims=True) + 1.0
    dinv = lax.rsqrt(deg)
    xs_ref[...] = xw_ref[...] * dinv
    dinv_ref[...] = dinv


def _prep(degt, xw):
    n, d = xw.shape
    return pl.pallas_call(
        _prep_body,
        grid=(n // RB,),
        in_specs=[
            pl.BlockSpec((RB, NC), lambda i: (i, 0)),
            pl.BlockSpec((RB, d), lambda i: (i, 0)),
        ],
        out_specs=[
            pl.BlockSpec((RB, d), lambda i: (i, 0)),
            pl.BlockSpec((RB, 1), lambda i: (i, 0)),
        ],
        out_shape=[
            jax.ShapeDtypeStruct((n, d), jnp.float32),
            jax.ShapeDtypeStruct((n, 1), jnp.float32),
        ],
    )(degt, xw)


SPLIT = 5040
GPAD = 64


NBUF = 5


def _make_mp_kernel(n, d, e):
    nch = e // (NW * CH)
    assert nch % NBUF == 0
    acc_rows = SPLIT + CH
    nzc = acc_rows // CH
    kmax = pl.cdiv(nzc, NS)
    c0chunks = SPLIT // CH
    mesh = plsc.VectorSubcoreMesh(core_axis_name="c", subcore_axis_name="s")

    @functools.partial(
        pl.kernel,
        out_type=jax.ShapeDtypeStruct((n, d), jnp.float32),
        mesh=mesh,
        scratch_types=[
            pltpu.VMEM((nch, CH), jnp.int32),
            pltpu.VMEM((nch, CH), jnp.int32),
            [pltpu.VMEM((CH, d), jnp.float32) for _ in range(NBUF)],
            [pltpu.SemaphoreType.DMA for _ in range(NBUF)],
            pltpu.VMEM_SHARED((acc_rows, d), jnp.float32),
        ],
    )
    def mp_kernel(xs_hbm, src_hbm, dst_hbm, out_hbm,
                  srcv, dstv, rows, sems, acc_sp):
        c = lax.axis_index("c")
        s = lax.axis_index("s")
        rows0 = rows[0]

        lo = c * SPLIT
        hi = SPLIT + c * (n - SPLIT)

        def zb(r, carry):
            for k2 in range(d // 16):
                rows0[r, pl.ds(k2 * 16, 16)] = jnp.zeros((16,), jnp.float32)
            return carry

        lax.fori_loop(0, CH, zb, 0)
        for k in range(kmax):
            cid = s + NS * k

            @pl.when(cid < nzc)
            def _():
                pltpu.sync_copy(rows0, acc_sp.at[pl.ds(cid * CH, CH)])

        for t in range(NC):
            wid = s * NC + t
            pltpu.sync_copy(src_hbm.at[wid], srcv)
            pltpu.sync_copy(dst_hbm.at[wid], dstv)

            def tf(j, carry):
                for k2 in range(CH // 16):
                    v = dstv[j, pl.ds(k2 * 16, 16)]
                    m = (v >= lo) & (v < hi)
                    g = SPLIT + (v & (GPAD - 1))
                    dstv[j, pl.ds(k2 * 16, 16)] = jnp.where(m, v - lo, g)
                return carry

            lax.fori_loop(0, nch, tf, 0)

            for k in range(NBUF - 1):
                pltpu.async_copy(xs_hbm.at[srcv.at[k]], rows[k], sems[k])

            def body(i, carry):
                for k in range(NBUF):
                    j = NBUF * i + k
                    pltpu.make_async_copy(xs_hbm.at[srcv.at[j]], rows[k],
                                          sems[k]).wait()
                    pltpu.sync_copy(rows[k], acc_sp.at[dstv.at[j]],
                                    add=True)
                    nxt = j + NBUF - 1
                    kn = (k + NBUF - 1) % NBUF

                    @pl.when(nxt < nch)
                    def _():
                        pltpu.async_copy(xs_hbm.at[srcv.at[nxt]], rows[kn],
                                         sems[kn])
                return carry

            lax.fori_loop(0, nch // NBUF, body, 0)

        plsc.subcore_barrier()
        for k in range(kmax):
            cid = s + NS * k

            @pl.when(cid < c0chunks - c)
            def _():
                pltpu.sync_copy(acc_sp.at[pl.ds(cid * CH, CH)], rows0)
                pltpu.sync_copy(rows0,
                                out_hbm.at[pl.ds(lo + cid * CH, CH)])

    return mp_kernel


def _final_body(h_ref, acc_ref, xs_ref, dinv_ref, bg_ref, g_ref, b_ref,
                out_ref):
    h = h_ref[...]
    t = acc_ref[...] + xs_ref[...]
    h2 = jnp.maximum(t * dinv_ref[...] + bg_ref[...], 0.0)
    cat = jnp.concatenate([h, h2], axis=1)
    mu = jnp.mean(cat, axis=1, keepdims=True)
    xc = cat - mu
    var = jnp.mean(xc * xc, axis=1, keepdims=True)
    out_ref[...] = xc * lax.rsqrt(var + 1e-5) * g_ref[...] + b_ref[...]


def _final(h, acc, xs, dinv, bg, gamma, beta):
    n, d = h.shape
    return pl.pallas_call(
        _final_body,
        grid=(n // RB,),
        in_specs=[
            pl.BlockSpec((RB, d), lambda i: (i, 0)),
            pl.BlockSpec((RB, d), lambda i: (i, 0)),
            pl.BlockSpec((RB, d), lambda i: (i, 0)),
            pl.BlockSpec((RB, 1), lambda i: (i, 0)),
            pl.BlockSpec((1, d), lambda i: (0, 0)),
            pl.BlockSpec((1, 2 * d), lambda i: (0, 0)),
            pl.BlockSpec((1, 2 * d), lambda i: (0, 0)),
        ],
        out_specs=pl.BlockSpec((RB, 2 * d), lambda i: (i, 0)),
        out_shape=jax.ShapeDtypeStruct((n, 2 * d), jnp.float32),
    )(h, acc, xs, dinv, bg.reshape(1, d), gamma.reshape(1, 2 * d),
      beta.reshape(1, 2 * d))


def kernel(x, edge_index, W1, b1, Wg, bg, gamma, beta):
    n, _ = x.shape
    d = W1.shape[1]
    e = edge_index.shape[1]
    nch = e // (NW * CH)
    src3 = edge_index[0].reshape(NW, nch, CH)
    dst3 = edge_index[1].reshape(NW, nch, CH)

    h, xw = _matmuls(x, W1, b1, Wg)
    degs = _make_deg_kernel(n, e)(dst3).reshape(NC, n)
    xs, dinv = _prep(degs.T, xw)
    acc = _make_mp_kernel(n, d, e)(xs, src3, dst3)
    return _final(h, acc, xs, dinv, bg, gamma, beta)
